# Initial kernel scaffold; baseline (speedup 1.0000x reference)
import jax
import jax.numpy as jnp
from jax.experimental import pallas as pl


def kernel(h, edge, W0, b0, wL, wR, Wk, bk, Wc, bias):
    raise NotImplementedError("write your pallas kernel here")


# speedup vs baseline: 11.2430x; 11.2430x over previous
import functools

import jax
import jax.numpy as jnp
from jax import lax
from jax.experimental import pallas as pl
from jax.experimental.pallas import tpu as pltpu
from jax.experimental.pallas import tpu_sc as plsc

_DELTA = 10.0
_LEAK = 0.01
_NC = 2
_NS = 16
_NW = _NC * _NS
_CHUNK = 128


def _sc_gather(table, idx3):
    nw, ch, lch = idx3.shape
    epw = ch * lch
    c = table.shape[1]
    grp = 8
    mesh = plsc.VectorSubcoreMesh(core_axis_name="c", subcore_axis_name="s")

    @functools.partial(
        pl.kernel,
        out_type=jax.ShapeDtypeStruct((nw, epw, c), jnp.float32),
        mesh=mesh,
        scratch_types=[
            pltpu.VMEM((ch, lch), jnp.int32),
            pltpu.VMEM((epw, c), jnp.float32),
            pltpu.SemaphoreType.DMA,
        ],
        compiler_params=pltpu.CompilerParams(use_tc_tiling_on_sc=False),
    )
    def body(table_hbm, idx_hbm, out_hbm, idx_v, rows_v, sem):
        wid = lax.axis_index("s") * _NC + lax.axis_index("c")
        pltpu.sync_copy(idx_hbm.at[wid], idx_v)

        def group(g, carry):
            handles = []
            for j in range(grp):
                cidx = g * grp + j
                handles.append(
                    pltpu.async_copy(
                        table_hbm.at[idx_v.at[cidx]],
                        rows_v.at[pl.ds(cidx * lch, lch)],
                        sem,
                    )
                )
            for hnd in handles:
                hnd.wait()
            return carry

        lax.fori_loop(0, ch // grp, group, 0)
        pltpu.sync_copy(rows_v, out_hbm.at[wid])

    return body(table, idx3)


def _tc_call(vertex4, center3, hp, w0t, wrp, wlf, e1, wkt, s_red, e2, s2, wc2,
             b0r, bkr, biasr, *, kk, nb, t, c, cout, rank, rows, wimg,
             interpret=False):
    def body(v_ref, c_ref, hp_ref, w0_ref, wrp_ref, wl_ref, e1_ref, wk_ref,
             s_ref, e2_ref, s2_ref, wc_ref, b0_ref, bk_ref, bias_ref, o_ref):
        ib = pl.program_id(0)
        ctr = c_ref[0]
        w0_ = w0_ref[...]
        wrp_ = wrp_ref[...]
        wl_ = wl_ref[...]
        e1_ = e1_ref[...]
        wk_ = wk_ref[...]
        s_ = s_ref[...]
        e2_ = e2_ref[...]
        s2_ = s2_ref[...]
        b0_ = b0_ref[...]
        bk_ = bk_ref[...]
        acc = jnp.zeros((t, cout), jnp.float32)
        for k in range(kk):
            v = v_ref[k, 0]
            lab = v - ctr
            pre = jnp.dot(lab, w0_, preferred_element_type=jnp.float32) + b0_
            theta = jnp.where(pre >= 0, pre, _LEAK * pre)
            g = jnp.dot(v, wrp_, preferred_element_type=jnp.float32)
            a_ = jnp.dot(theta, wl_, preferred_element_type=jnp.float32)
            th_e = jnp.dot(theta, e1_, preferred_element_type=jnp.float32)
            kap = jnp.dot(theta, wk_, preferred_element_type=jnp.float32) + bk_
            tmp = jnp.dot(th_e * g, s_, preferred_element_type=jnp.float32)
            ssq = jnp.sum(lab * lab, axis=1, keepdims=True)
            gam = jnp.exp(ssq * (-1.0 / _DELTA))
            sv = kap * tmp * gam
            s_e = jnp.dot(sv, e2_, preferred_element_type=jnp.float32)
            acc = acc + jnp.dot(a_ * s_e, s2_, preferred_element_type=jnp.float32)
        taps = []
        for dy in range(3):
            for dx in range(3):
                blk = hp_ref[pl.ds(ib * rows + dy, rows), pl.ds(dx, wimg), :]
                taps.append(blk.reshape(t, c))
        hcat = jnp.concatenate(taps, axis=1)
        h_l = jnp.dot(hcat, wc_ref[...], preferred_element_type=jnp.float32)
        o_ref[0] = acc * (0.5 / kk) + h_l * 0.5 + bias_ref[...]

    full = lambda a: pl.BlockSpec(a.shape, lambda i: (0,) * a.ndim)
    return pl.pallas_call(
        body,
        grid=(nb,),
        in_specs=[
            pl.BlockSpec((kk, 1, t, c), lambda i: (0, i, 0, 0)),
            pl.BlockSpec((1, t, c), lambda i: (i, 0, 0)),
            full(hp), full(w0t), full(wrp), full(wlf), full(e1), full(wkt),
            full(s_red), full(e2), full(s2), full(wc2), full(b0r), full(bkr),
            full(biasr),
        ],
        out_specs=pl.BlockSpec((1, t, cout), lambda i: (i, 0, 0)),
        out_shape=jax.ShapeDtypeStruct((nb, t, cout), jnp.float32),
        interpret=interpret,
    )(vertex4, center3, hp, w0t, wrp, wlf, e1, wkt, s_red, e2, s2, wc2,
      b0r, bkr, biasr)


def _weights(W0, b0, wL, wR, Wk, bk, Wc, bias, c, cout, rank):
    m = (rank * cout) // wL.shape[0]
    cols = jnp.arange(rank * c)
    irow = jnp.arange(c)

    def circfull(w):
        wm = w[:, 0, :]
        return wm[cols[None, :] // m, (irow[:, None] - cols[None, :] % m) % c]

    wlf = circfull(wL)
    wrf = circfull(wR)
    wrp = wrf.reshape(c, c, rank).transpose(1, 0, 2).reshape(c, c * rank)
    e1 = (cols[None, :] // rank == jnp.arange(c)[:, None]).astype(jnp.float32)
    s_red = (cols[:, None] % rank == jnp.arange(rank)[None, :]).astype(jnp.float32)
    e2 = (cols[None, :] % rank == jnp.arange(rank)[:, None]).astype(jnp.float32)
    s2 = (cols[:, None] // rank == jnp.arange(cout)[None, :]).astype(jnp.float32)
    wc2 = Wc.transpose(2, 3, 1, 0).reshape(9 * c, cout)
    return (W0.T, wrp, wlf, e1, Wk.T, s_red, e2, s2, wc2,
            b0.reshape(1, c), bk.reshape(1, rank), bias.reshape(1, cout))


def kernel(h, edge, W0, b0, wL, wR, Wk, bk, Wc, bias):
    b, c, himg, wimg = h.shape
    kk = edge.shape[1]
    n = himg * wimg
    cout = Wc.shape[0]
    rank = Wk.shape[0]
    e_tot = kk * n

    table = h.reshape(c, n).T
    epw = e_tot // _NW
    idx3 = edge.reshape(_NW, epw // _CHUNK, _CHUNK)
    vertex = _sc_gather(table, idx3)

    rows = 16
    t = rows * wimg
    nb = n // t
    vertex4 = vertex.reshape(kk, nb, t, c)
    center3 = table.reshape(nb, t, c)
    hp = jnp.pad(h[0], ((0, 0), (1, 1), (1, 1)), mode="reflect").transpose(1, 2, 0)

    ws = _weights(W0, b0, wL, wR, Wk, bk, Wc, bias, c, cout, rank)
    out_pm = _tc_call(vertex4, center3, hp, *ws, kk=kk, nb=nb, t=t, c=c,
                      cout=cout, rank=rank, rows=rows, wimg=wimg)
    return out_pm.reshape(n, cout).T.reshape(b, cout, himg, wimg)
